# Initial kernel scaffold; baseline (speedup 1.0000x reference)
import jax
import jax.numpy as jnp
from jax.experimental import pallas as pl


def kernel(x, edge_index, batch, W_rel1, W_root1, b1, g1, be1, W_rel2, W_root2, b2, g2, be2, W_rel3, W_root3, b3, g3, be3):
    raise NotImplementedError("write your pallas kernel here")


# speedup vs baseline: 7.6812x; 7.6812x over previous
import functools

import jax
import jax.numpy as jnp
from jax import lax
from jax.experimental import pallas as pl
from jax.experimental.pallas import tpu as pltpu
from jax.experimental.pallas import tpu_sc as plsc

N = 10000
E = 320000
G = 64
EPS = 1e-5

NC = 2
NS = 16
NW = NC * NS
EPW = E // NW
CHUNK = 125
NCHUNK = EPW // CHUNK
RPT = 632
NPAD = RPT * NS
LAST = N - RPT * (NS - 1)

ROWS = 2000
NBLK = N // ROWS


def _make_edge_agg(D):
    mesh = plsc.VectorSubcoreMesh(
        core_axis_name="c", subcore_axis_name="s",
        num_cores=NC, num_subcores=NS)

    @functools.partial(
        pl.kernel,
        out_type=jax.ShapeDtypeStruct((NC, N, D), jnp.float32),
        mesh=mesh,
        scratch_types=[
            pltpu.VMEM((NCHUNK, CHUNK), jnp.int32),
            pltpu.VMEM((NCHUNK, CHUNK), jnp.int32),
            pltpu.VMEM((CHUNK, D), jnp.float32),
            pltpu.VMEM_SHARED((NPAD, D), jnp.float32),
            pltpu.SemaphoreType.DMA,
        ],
    )
    def edge_agg(y_hbm, src_hbm, dst_hbm, zero_hbm, out_hbm,
                 sidx, didx, rows, acc, sem):
        cid = lax.axis_index("c")
        sid = lax.axis_index("s")
        wid = sid * NC + cid
        pltpu.sync_copy(src_hbm.at[wid], sidx)
        pltpu.sync_copy(dst_hbm.at[wid], didx)
        pltpu.sync_copy(zero_hbm.at[pl.ds(sid * RPT, RPT)],
                        acc.at[pl.ds(sid * RPT, RPT)])
        plsc.subcore_barrier()

        def step(i, carry):
            pltpu.async_copy(y_hbm.at[sidx.at[i]], rows, sem).wait()
            pltpu.sync_copy(rows, acc.at[didx.at[i]], add=True)
            return carry

        lax.fori_loop(0, NCHUNK, step, 0)
        plsc.subcore_barrier()

        @pl.when(sid < NS - 1)
        def _():
            pltpu.sync_copy(acc.at[pl.ds(sid * RPT, RPT)],
                            out_hbm.at[cid, pl.ds(sid * RPT, RPT)])

        @pl.when(sid == NS - 1)
        def _():
            pltpu.sync_copy(acc.at[pl.ds((NS - 1) * RPT, LAST)],
                            out_hbm.at[cid, pl.ds((NS - 1) * RPT, LAST)])

    return edge_agg


_make_edge_agg = functools.lru_cache(maxsize=None)(_make_edge_agg)


def _mm_body(x_ref, w_ref, o_ref):
    o_ref[...] = jnp.dot(x_ref[...], w_ref[...],
                         preferred_element_type=jnp.float32)


def _mm(x, w):
    din, dout = w.shape
    return pl.pallas_call(
        _mm_body,
        grid=(NBLK,),
        in_specs=[pl.BlockSpec((ROWS, din), lambda i: (i, 0)),
                  pl.BlockSpec((din, dout), lambda i: (0, 0))],
        out_specs=pl.BlockSpec((ROWS, dout), lambda i: (i, 0)),
        out_shape=jax.ShapeDtypeStruct((N, dout), jnp.float32),
    )(x, w)


def _norm_relu(a, h_ref, wroot_ref, b_ref, g_ref, be_ref):
    z = (a
         + jnp.dot(h_ref[...], wroot_ref[...],
                   preferred_element_type=jnp.float32)
         + b_ref[...])
    mu = jnp.mean(z, axis=1, keepdims=True)
    zc = z - mu
    var = jnp.mean(zc * zc, axis=1, keepdims=True)
    return jnp.maximum(zc * lax.rsqrt(var + EPS) * g_ref[...] + be_ref[...],
                       0.0)


def _layer_body(agg_ref, h_ref, wroot_ref, b_ref, g_ref, be_ref, wnext_ref,
                h_out, y_out):
    h = _norm_relu(agg_ref[0] + agg_ref[1], h_ref, wroot_ref, b_ref, g_ref,
                   be_ref)
    h_out[...] = h
    y_out[...] = jnp.dot(h, wnext_ref[...], preferred_element_type=jnp.float32)


def _layer(agg, h_prev, wroot, b, g, be, wnext):
    d = wroot.shape[1]
    dnext = wnext.shape[1]
    return pl.pallas_call(
        _layer_body,
        grid=(NBLK,),
        in_specs=[pl.BlockSpec((NC, ROWS, d), lambda i: (0, i, 0)),
                  pl.BlockSpec((ROWS, h_prev.shape[1]), lambda i: (i, 0)),
                  pl.BlockSpec(wroot.shape, lambda i: (0, 0)),
                  pl.BlockSpec((1, d), lambda i: (0, 0)),
                  pl.BlockSpec((1, d), lambda i: (0, 0)),
                  pl.BlockSpec((1, d), lambda i: (0, 0)),
                  pl.BlockSpec(wnext.shape, lambda i: (0, 0))],
        out_specs=[pl.BlockSpec((ROWS, d), lambda i: (i, 0)),
                   pl.BlockSpec((ROWS, dnext), lambda i: (i, 0))],
        out_shape=[jax.ShapeDtypeStruct((N, d), jnp.float32),
                   jax.ShapeDtypeStruct((N, dnext), jnp.float32)],
    )(agg, h_prev, wroot, b, g, be, wnext)


def _final_body(agg_ref, h_ref, wroot_ref, b_ref, g_ref, be_ref, batch_ref,
                o_ref, cnt_ref):
    i = pl.program_id(0)
    a = (agg_ref[0] + agg_ref[1])[:, :wroot_ref.shape[1]]
    h = _norm_relu(a, h_ref, wroot_ref, b_ref, g_ref, be_ref)
    gids = lax.broadcasted_iota(jnp.int32, (ROWS, G), 1)
    onehot = (batch_ref[...] == gids).astype(jnp.float32)
    sums = lax.dot_general(onehot, h, (((0,), (0,)), ((), ())),
                           preferred_element_type=jnp.float32)
    cnts = lax.dot_general(onehot, jnp.ones((ROWS, 1), jnp.float32),
                           (((0,), (0,)), ((), ())),
                           preferred_element_type=jnp.float32)

    @pl.when(i == 0)
    def _():
        o_ref[...] = jnp.zeros_like(o_ref)
        cnt_ref[...] = jnp.zeros_like(cnt_ref)

    o_ref[...] += sums
    cnt_ref[...] += cnts

    @pl.when(i == NBLK - 1)
    def _():
        o_ref[...] = o_ref[...] / jnp.maximum(cnt_ref[...], 1.0)


def _final(agg, h_prev, wroot, b, g, be, batch2d):
    d = wroot.shape[1]
    return pl.pallas_call(
        _final_body,
        grid=(NBLK,),
        in_specs=[pl.BlockSpec((NC, ROWS, 128), lambda i: (0, i, 0)),
                  pl.BlockSpec((ROWS, h_prev.shape[1]), lambda i: (i, 0)),
                  pl.BlockSpec(wroot.shape, lambda i: (0, 0)),
                  pl.BlockSpec((1, d), lambda i: (0, 0)),
                  pl.BlockSpec((1, d), lambda i: (0, 0)),
                  pl.BlockSpec((1, d), lambda i: (0, 0)),
                  pl.BlockSpec((ROWS, 1), lambda i: (i, 0))],
        out_specs=pl.BlockSpec((G, d), lambda i: (0, 0)),
        out_shape=jax.ShapeDtypeStruct((G, d), jnp.float32),
        scratch_shapes=[pltpu.VMEM((G, 1), jnp.float32)],
    )(agg, h_prev, wroot, b, g, be, batch2d)


def kernel(x, edge_index, batch,
           W_rel1, W_root1, b1, g1, be1,
           W_rel2, W_root2, b2, g2, be2,
           W_rel3, W_root3, b3, g3, be3):
    src_r = edge_index[0].astype(jnp.int32).reshape(NW, NCHUNK, CHUNK)
    dst_r = edge_index[1].astype(jnp.int32).reshape(NW, NCHUNK, CHUNK)
    z128 = jnp.zeros((NPAD, 128), jnp.float32)
    batch2d = batch.astype(jnp.int32).reshape(N, 1)
    b1r, g1r, be1r = b1.reshape(1, -1), g1.reshape(1, -1), be1.reshape(1, -1)
    b2r, g2r, be2r = b2.reshape(1, -1), g2.reshape(1, -1), be2.reshape(1, -1)
    b3r, g3r, be3r = b3.reshape(1, -1), g3.reshape(1, -1), be3.reshape(1, -1)

    W_rel3p = jnp.zeros((W_rel3.shape[0], 128), jnp.float32).at[:, :W_rel3.shape[1]].set(W_rel3)

    y1 = _mm(x, W_rel1)
    a1 = _make_edge_agg(128)(y1, src_r, dst_r, z128)
    h1, y2 = _layer(a1, x, W_root1, b1r, g1r, be1r, W_rel2)
    a2 = _make_edge_agg(128)(y2, src_r, dst_r, z128)
    h2, y3 = _layer(a2, h1, W_root2, b2r, g2r, be2r, W_rel3p)
    a3 = _make_edge_agg(128)(y3, src_r, dst_r, z128)
    return _final(a3, h2, W_root3, b3r, g3r, be3r, batch2d)


# speedup vs baseline: 9.2097x; 1.1990x over previous
import functools

import jax
import jax.numpy as jnp
from jax import lax
from jax.experimental import pallas as pl
from jax.experimental.pallas import tpu as pltpu
from jax.experimental.pallas import tpu_sc as plsc

N = 10000
E = 320000
G = 64
EPS = 1e-5

NC = 2
NS = 16
NW = NC * NS
EPW = E // NW
CHUNK = 100
NCHUNK = EPW // CHUNK
RPT = 632
NPAD = RPT * NS
LAST = N - RPT * (NS - 1)

ROWS = 2000
NBLK = N // ROWS


def _make_edge_agg(D):
    mesh = plsc.VectorSubcoreMesh(
        core_axis_name="c", subcore_axis_name="s",
        num_cores=NC, num_subcores=NS)

    @functools.partial(
        pl.kernel,
        out_type=jax.ShapeDtypeStruct((NC, N, D), jnp.float32),
        mesh=mesh,
        compiler_params=pltpu.CompilerParams(use_tc_tiling_on_sc=False),
        scratch_types=[
            pltpu.VMEM((NCHUNK, CHUNK), jnp.int32),
            pltpu.VMEM((NCHUNK, CHUNK), jnp.int32),
            pltpu.VMEM((2, CHUNK, D), jnp.float32),
            pltpu.VMEM_SHARED((NPAD, D), jnp.float32),
            pltpu.SemaphoreType.DMA((2,)),
        ],
    )
    def edge_agg(y_hbm, src_hbm, dst_hbm, zero_hbm, out_hbm,
                 sidx, didx, rows, acc, sem):
        cid = lax.axis_index("c")
        sid = lax.axis_index("s")
        wid = sid * NC + cid
        pltpu.sync_copy(src_hbm.at[wid], sidx)
        pltpu.sync_copy(dst_hbm.at[wid], didx)
        pltpu.sync_copy(zero_hbm.at[pl.ds(sid * RPT, RPT)],
                        acc.at[pl.ds(sid * RPT, RPT)])
        plsc.subcore_barrier()

        pltpu.async_copy(y_hbm.at[sidx.at[0]], rows.at[0], sem.at[0])

        def step(i, carry):
            b = lax.rem(i, 2)
            nb = 1 - b
            nxt = lax.rem(i + 1, NCHUNK)
            pltpu.make_async_copy(y_hbm.at[sidx.at[i]], rows.at[b],
                                  sem.at[b]).wait()
            pltpu.async_copy(y_hbm.at[sidx.at[nxt]], rows.at[nb], sem.at[nb])
            pltpu.sync_copy(rows.at[b], acc.at[didx.at[i]], add=True)
            return carry

        lax.fori_loop(0, NCHUNK, step, 0)
        pltpu.make_async_copy(y_hbm.at[sidx.at[0]], rows.at[0],
                              sem.at[NCHUNK % 2]).wait()
        plsc.subcore_barrier()

        @pl.when(sid < NS - 1)
        def _():
            pltpu.sync_copy(acc.at[pl.ds(sid * RPT, RPT)],
                            out_hbm.at[cid, pl.ds(sid * RPT, RPT)])

        @pl.when(sid == NS - 1)
        def _():
            pltpu.sync_copy(acc.at[pl.ds((NS - 1) * RPT, LAST)],
                            out_hbm.at[cid, pl.ds((NS - 1) * RPT, LAST)])

    return edge_agg


_make_edge_agg = functools.lru_cache(maxsize=None)(_make_edge_agg)


def _mm_body(x_ref, w_ref, o_ref):
    o_ref[...] = jnp.dot(x_ref[...], w_ref[...],
                         preferred_element_type=jnp.float32)


def _mm(x, w):
    din, dout = w.shape
    return pl.pallas_call(
        _mm_body,
        grid=(NBLK,),
        in_specs=[pl.BlockSpec((ROWS, din), lambda i: (i, 0)),
                  pl.BlockSpec((din, dout), lambda i: (0, 0))],
        out_specs=pl.BlockSpec((ROWS, dout), lambda i: (i, 0)),
        out_shape=jax.ShapeDtypeStruct((N, dout), jnp.float32),
    )(x, w)


def _norm_relu(a, h_ref, wroot_ref, b_ref, g_ref, be_ref):
    z = (a
         + jnp.dot(h_ref[...], wroot_ref[...],
                   preferred_element_type=jnp.float32)
         + b_ref[...])
    mu = jnp.mean(z, axis=1, keepdims=True)
    zc = z - mu
    var = jnp.mean(zc * zc, axis=1, keepdims=True)
    return jnp.maximum(zc * lax.rsqrt(var + EPS) * g_ref[...] + be_ref[...],
                       0.0)


def _layer_body(agg_ref, h_ref, wroot_ref, b_ref, g_ref, be_ref, wnext_ref,
                h_out, y_out):
    h = _norm_relu(agg_ref[0] + agg_ref[1], h_ref, wroot_ref, b_ref, g_ref,
                   be_ref)
    h_out[...] = h
    y_out[...] = jnp.dot(h, wnext_ref[...], preferred_element_type=jnp.float32)


def _layer(agg, h_prev, wroot, b, g, be, wnext):
    d = wroot.shape[1]
    dnext = wnext.shape[1]
    return pl.pallas_call(
        _layer_body,
        grid=(NBLK,),
        in_specs=[pl.BlockSpec((NC, ROWS, d), lambda i: (0, i, 0)),
                  pl.BlockSpec((ROWS, h_prev.shape[1]), lambda i: (i, 0)),
                  pl.BlockSpec(wroot.shape, lambda i: (0, 0)),
                  pl.BlockSpec((1, d), lambda i: (0, 0)),
                  pl.BlockSpec((1, d), lambda i: (0, 0)),
                  pl.BlockSpec((1, d), lambda i: (0, 0)),
                  pl.BlockSpec(wnext.shape, lambda i: (0, 0))],
        out_specs=[pl.BlockSpec((ROWS, d), lambda i: (i, 0)),
                   pl.BlockSpec((ROWS, dnext), lambda i: (i, 0))],
        out_shape=[jax.ShapeDtypeStruct((N, d), jnp.float32),
                   jax.ShapeDtypeStruct((N, dnext), jnp.float32)],
    )(agg, h_prev, wroot, b, g, be, wnext)


def _final_body(agg_ref, h_ref, wroot_ref, b_ref, g_ref, be_ref, batch_ref,
                o_ref, cnt_ref):
    i = pl.program_id(0)
    a = (agg_ref[0] + agg_ref[1])[:, :wroot_ref.shape[1]]
    h = _norm_relu(a, h_ref, wroot_ref, b_ref, g_ref, be_ref)
    gids = lax.broadcasted_iota(jnp.int32, (ROWS, G), 1)
    onehot = (batch_ref[...] == gids).astype(jnp.float32)
    sums = lax.dot_general(onehot, h, (((0,), (0,)), ((), ())),
                           preferred_element_type=jnp.float32)
    cnts = lax.dot_general(onehot, jnp.ones((ROWS, 1), jnp.float32),
                           (((0,), (0,)), ((), ())),
                           preferred_element_type=jnp.float32)

    @pl.when(i == 0)
    def _():
        o_ref[...] = jnp.zeros_like(o_ref)
        cnt_ref[...] = jnp.zeros_like(cnt_ref)

    o_ref[...] += sums
    cnt_ref[...] += cnts

    @pl.when(i == NBLK - 1)
    def _():
        o_ref[...] = o_ref[...] / jnp.maximum(cnt_ref[...], 1.0)


def _final(agg, h_prev, wroot, b, g, be, batch2d):
    d = wroot.shape[1]
    return pl.pallas_call(
        _final_body,
        grid=(NBLK,),
        in_specs=[pl.BlockSpec((NC, ROWS, 128), lambda i: (0, i, 0)),
                  pl.BlockSpec((ROWS, h_prev.shape[1]), lambda i: (i, 0)),
                  pl.BlockSpec(wroot.shape, lambda i: (0, 0)),
                  pl.BlockSpec((1, d), lambda i: (0, 0)),
                  pl.BlockSpec((1, d), lambda i: (0, 0)),
                  pl.BlockSpec((1, d), lambda i: (0, 0)),
                  pl.BlockSpec((ROWS, 1), lambda i: (i, 0))],
        out_specs=pl.BlockSpec((G, d), lambda i: (0, 0)),
        out_shape=jax.ShapeDtypeStruct((G, d), jnp.float32),
        scratch_shapes=[pltpu.VMEM((G, 1), jnp.float32)],
    )(agg, h_prev, wroot, b, g, be, batch2d)


def kernel(x, edge_index, batch,
           W_rel1, W_root1, b1, g1, be1,
           W_rel2, W_root2, b2, g2, be2,
           W_rel3, W_root3, b3, g3, be3):
    src_r = edge_index[0].astype(jnp.int32).reshape(NW, NCHUNK, CHUNK)
    dst_r = edge_index[1].astype(jnp.int32).reshape(NW, NCHUNK, CHUNK)
    z128 = jnp.zeros((NPAD, 128), jnp.float32)
    batch2d = batch.astype(jnp.int32).reshape(N, 1)
    b1r, g1r, be1r = b1.reshape(1, -1), g1.reshape(1, -1), be1.reshape(1, -1)
    b2r, g2r, be2r = b2.reshape(1, -1), g2.reshape(1, -1), be2.reshape(1, -1)
    b3r, g3r, be3r = b3.reshape(1, -1), g3.reshape(1, -1), be3.reshape(1, -1)

    W_rel3p = jnp.zeros((W_rel3.shape[0], 128), jnp.float32).at[:, :W_rel3.shape[1]].set(W_rel3)

    y1 = _mm(x, W_rel1)
    a1 = _make_edge_agg(128)(y1, src_r, dst_r, z128)
    h1, y2 = _layer(a1, x, W_root1, b1r, g1r, be1r, W_rel2)
    a2 = _make_edge_agg(128)(y2, src_r, dst_r, z128)
    h2, y3 = _layer(a2, h1, W_root2, b2r, g2r, be2r, W_rel3p)
    a3 = _make_edge_agg(128)(y3, src_r, dst_r, z128)
    return _final(a3, h2, W_root3, b3r, g3r, be3r, batch2d)


# speedup vs baseline: 9.6871x; 1.0518x over previous
import functools

import jax
import jax.numpy as jnp
from jax import lax
from jax.experimental import pallas as pl
from jax.experimental.pallas import tpu as pltpu
from jax.experimental.pallas import tpu_sc as plsc

N = 10000
E = 320000
G = 64
EPS = 1e-5

NC = 2
NS = 16
NW = NC * NS
EPW = E // NW
CHUNK = 100
NCHUNK = EPW // CHUNK
RPT = 632
NPAD = RPT * NS
LAST = N - RPT * (NS - 1)

ROWS = 2000
NBLK = N // ROWS


def _make_edge_agg(D):
    mesh = plsc.VectorSubcoreMesh(
        core_axis_name="c", subcore_axis_name="s",
        num_cores=NC, num_subcores=NS)

    @functools.partial(
        pl.kernel,
        out_type=jax.ShapeDtypeStruct((NC, N, D), jnp.float32),
        mesh=mesh,
        compiler_params=pltpu.CompilerParams(use_tc_tiling_on_sc=False),
        scratch_types=[
            pltpu.VMEM((NCHUNK, CHUNK), jnp.int32),
            pltpu.VMEM((NCHUNK, CHUNK), jnp.int32),
            pltpu.VMEM((2, CHUNK, D), jnp.float32),
            pltpu.VMEM_SHARED((NPAD, D), jnp.float32),
            pltpu.SemaphoreType.DMA((2,)),
        ],
    )
    def edge_agg(y_hbm, src_hbm, dst_hbm, zero_hbm, out_hbm,
                 sidx, didx, rows, acc, sem):
        cid = lax.axis_index("c")
        sid = lax.axis_index("s")
        wid = sid * NC + cid
        pltpu.sync_copy(src_hbm.at[wid], sidx)
        pltpu.sync_copy(dst_hbm.at[wid], didx)
        pltpu.sync_copy(zero_hbm.at[pl.ds(sid * RPT, RPT)],
                        acc.at[pl.ds(sid * RPT, RPT)])
        plsc.subcore_barrier()

        pltpu.async_copy(y_hbm.at[sidx.at[0]], rows.at[0], sem.at[0])

        def step(i, carry):
            b = lax.rem(i, 2)
            nb = 1 - b
            nxt = lax.rem(i + 1, NCHUNK)
            pltpu.make_async_copy(y_hbm.at[sidx.at[i]], rows.at[b],
                                  sem.at[b]).wait()
            pltpu.async_copy(y_hbm.at[sidx.at[nxt]], rows.at[nb], sem.at[nb])
            pltpu.sync_copy(rows.at[b], acc.at[didx.at[i]], add=True)
            return carry

        lax.fori_loop(0, NCHUNK, step, 0)
        pltpu.make_async_copy(y_hbm.at[sidx.at[0]], rows.at[0],
                              sem.at[NCHUNK % 2]).wait()
        plsc.subcore_barrier()

        @pl.when(sid < NS - 1)
        def _():
            pltpu.sync_copy(acc.at[pl.ds(sid * RPT, RPT)],
                            out_hbm.at[cid, pl.ds(sid * RPT, RPT)])

        @pl.when(sid == NS - 1)
        def _():
            pltpu.sync_copy(acc.at[pl.ds((NS - 1) * RPT, LAST)],
                            out_hbm.at[cid, pl.ds((NS - 1) * RPT, LAST)])

    return edge_agg


_make_edge_agg = functools.lru_cache(maxsize=None)(_make_edge_agg)


def _mm_body(x_ref, w_ref, o_ref):
    o_ref[...] = jnp.dot(x_ref[...], w_ref[...],
                         preferred_element_type=jnp.float32)


def _mm(x, w):
    din, dout = w.shape
    return pl.pallas_call(
        _mm_body,
        grid=(NBLK,),
        in_specs=[pl.BlockSpec((ROWS, din), lambda i: (i, 0)),
                  pl.BlockSpec((din, dout), lambda i: (0, 0))],
        out_specs=pl.BlockSpec((ROWS, dout), lambda i: (i, 0)),
        out_shape=jax.ShapeDtypeStruct((N, dout), jnp.float32),
    )(x, w)


def _norm_relu(a, h_ref, wroot_ref, b_ref, g_ref, be_ref):
    z = (a
         + jnp.dot(h_ref[...], wroot_ref[...],
                   preferred_element_type=jnp.float32)
         + b_ref[...])
    mu = jnp.mean(z, axis=1, keepdims=True)
    zc = z - mu
    var = jnp.mean(zc * zc, axis=1, keepdims=True)
    return jnp.maximum(zc * lax.rsqrt(var + EPS) * g_ref[...] + be_ref[...],
                       0.0)


def _layer_body(agg_ref, h_ref, wroot_ref, b_ref, g_ref, be_ref, wnext_ref,
                h_out, y_out):
    h = _norm_relu(agg_ref[0] + agg_ref[1], h_ref, wroot_ref, b_ref, g_ref,
                   be_ref)
    h_out[...] = h
    y_out[...] = jnp.dot(h, wnext_ref[...], preferred_element_type=jnp.float32)


def _layer(agg, h_prev, wroot, b, g, be, wnext):
    d = wroot.shape[1]
    dnext = wnext.shape[1]
    return pl.pallas_call(
        _layer_body,
        grid=(NBLK,),
        in_specs=[pl.BlockSpec((NC, ROWS, d), lambda i: (0, i, 0)),
                  pl.BlockSpec((ROWS, h_prev.shape[1]), lambda i: (i, 0)),
                  pl.BlockSpec(wroot.shape, lambda i: (0, 0)),
                  pl.BlockSpec((1, d), lambda i: (0, 0)),
                  pl.BlockSpec((1, d), lambda i: (0, 0)),
                  pl.BlockSpec((1, d), lambda i: (0, 0)),
                  pl.BlockSpec(wnext.shape, lambda i: (0, 0))],
        out_specs=[pl.BlockSpec((ROWS, d), lambda i: (i, 0)),
                   pl.BlockSpec((ROWS, dnext), lambda i: (i, 0))],
        out_shape=[jax.ShapeDtypeStruct((N, d), jnp.float32),
                   jax.ShapeDtypeStruct((N, dnext), jnp.float32)],
    )(agg, h_prev, wroot, b, g, be, wnext)


def _final_body(agg_ref, h_ref, wroot_ref, b_ref, g_ref, be_ref, batch_ref,
                o_ref, cnt_ref):
    i = pl.program_id(0)
    a = agg_ref[0] + agg_ref[1]
    h = _norm_relu(a, h_ref, wroot_ref, b_ref, g_ref, be_ref)
    gids = lax.broadcasted_iota(jnp.int32, (ROWS, G), 1)
    onehot = (batch_ref[...] == gids).astype(jnp.float32)
    sums = lax.dot_general(onehot, h, (((0,), (0,)), ((), ())),
                           preferred_element_type=jnp.float32)
    cnts = lax.dot_general(onehot, jnp.ones((ROWS, 1), jnp.float32),
                           (((0,), (0,)), ((), ())),
                           preferred_element_type=jnp.float32)

    @pl.when(i == 0)
    def _():
        o_ref[...] = jnp.zeros_like(o_ref)
        cnt_ref[...] = jnp.zeros_like(cnt_ref)

    o_ref[...] += sums
    cnt_ref[...] += cnts

    @pl.when(i == NBLK - 1)
    def _():
        o_ref[...] = o_ref[...] / jnp.maximum(cnt_ref[...], 1.0)


def _final(agg, h_prev, wroot, b, g, be, batch2d):
    d = wroot.shape[1]
    return pl.pallas_call(
        _final_body,
        grid=(NBLK,),
        in_specs=[pl.BlockSpec((NC, ROWS, d), lambda i: (0, i, 0)),
                  pl.BlockSpec((ROWS, h_prev.shape[1]), lambda i: (i, 0)),
                  pl.BlockSpec(wroot.shape, lambda i: (0, 0)),
                  pl.BlockSpec((1, d), lambda i: (0, 0)),
                  pl.BlockSpec((1, d), lambda i: (0, 0)),
                  pl.BlockSpec((1, d), lambda i: (0, 0)),
                  pl.BlockSpec((ROWS, 1), lambda i: (i, 0))],
        out_specs=pl.BlockSpec((G, d), lambda i: (0, 0)),
        out_shape=jax.ShapeDtypeStruct((G, d), jnp.float32),
        scratch_shapes=[pltpu.VMEM((G, 1), jnp.float32)],
    )(agg, h_prev, wroot, b, g, be, batch2d)


def kernel(x, edge_index, batch,
           W_rel1, W_root1, b1, g1, be1,
           W_rel2, W_root2, b2, g2, be2,
           W_rel3, W_root3, b3, g3, be3):
    src_r = edge_index[0].astype(jnp.int32).reshape(NW, NCHUNK, CHUNK)
    dst_r = edge_index[1].astype(jnp.int32).reshape(NW, NCHUNK, CHUNK)
    z128 = jnp.zeros((NPAD, 128), jnp.float32)
    batch2d = batch.astype(jnp.int32).reshape(N, 1)
    b1r, g1r, be1r = b1.reshape(1, -1), g1.reshape(1, -1), be1.reshape(1, -1)
    b2r, g2r, be2r = b2.reshape(1, -1), g2.reshape(1, -1), be2.reshape(1, -1)
    b3r, g3r, be3r = b3.reshape(1, -1), g3.reshape(1, -1), be3.reshape(1, -1)

    z64 = jnp.zeros((NPAD, 64), jnp.float32)

    y1 = _mm(x, W_rel1)
    a1 = _make_edge_agg(128)(y1, src_r, dst_r, z128)
    h1, y2 = _layer(a1, x, W_root1, b1r, g1r, be1r, W_rel2)
    a2 = _make_edge_agg(128)(y2, src_r, dst_r, z128)
    h2, y3 = _layer(a2, h1, W_root2, b2r, g2r, be2r, W_rel3)
    a3 = _make_edge_agg(64)(y3, src_r, dst_r, z64)
    return _final(a3, h2, W_root3, b3r, g3r, be3r, batch2d)


# speedup vs baseline: 9.6933x; 1.0006x over previous
import functools

import jax
import jax.numpy as jnp
from jax import lax
from jax.experimental import pallas as pl
from jax.experimental.pallas import tpu as pltpu
from jax.experimental.pallas import tpu_sc as plsc

N = 10000
E = 320000
G = 64
EPS = 1e-5

NC = 2
NS = 16
NW = NC * NS
EPW = E // NW
CHUNK = 100
NCHUNK = EPW // CHUNK
RPT = 632
NPAD = RPT * NS
LAST = N - RPT * (NS - 1)
NBUF = 2

ROWS = 2000
NBLK = N // ROWS


def _make_edge_agg(D):
    mesh = plsc.VectorSubcoreMesh(
        core_axis_name="c", subcore_axis_name="s",
        num_cores=NC, num_subcores=NS)

    @functools.partial(
        pl.kernel,
        out_type=jax.ShapeDtypeStruct((NC, N, D), jnp.float32),
        mesh=mesh,
        compiler_params=pltpu.CompilerParams(use_tc_tiling_on_sc=False),
        scratch_types=[
            pltpu.VMEM((NCHUNK, CHUNK), jnp.int32),
            pltpu.VMEM((NCHUNK, CHUNK), jnp.int32),
            pltpu.VMEM((NBUF, CHUNK, D), jnp.float32),
            pltpu.VMEM_SHARED((NPAD, D), jnp.float32),
            pltpu.SemaphoreType.DMA((NBUF,)),
            pltpu.SemaphoreType.DMA((NBUF,)),
        ],
    )
    def edge_agg(y_hbm, src_hbm, dst_hbm, zero_hbm, out_hbm,
                 sidx, didx, rows, acc, sem, sem_s):
        cid = lax.axis_index("c")
        sid = lax.axis_index("s")
        wid = sid * NC + cid
        pltpu.sync_copy(src_hbm.at[wid], sidx)
        pltpu.sync_copy(dst_hbm.at[wid], didx)
        pltpu.sync_copy(zero_hbm.at[pl.ds(sid * RPT, RPT)],
                        acc.at[pl.ds(sid * RPT, RPT)])
        plsc.subcore_barrier()

        pltpu.async_copy(y_hbm.at[sidx.at[0]], rows.at[0], sem.at[0])

        def step(i, carry):
            b = lax.rem(i, NBUF)
            nb = lax.rem(i + 1, NBUF)

            @pl.when(i >= 1)
            def _():
                pltpu.make_async_copy(rows.at[nb], acc.at[didx.at[i - 1]],
                                      sem_s.at[nb]).wait()

            pltpu.make_async_copy(y_hbm.at[sidx.at[i]], rows.at[b],
                                  sem.at[b]).wait()
            nxt = lax.rem(i + 1, NCHUNK)
            pltpu.async_copy(y_hbm.at[sidx.at[nxt]], rows.at[nb], sem.at[nb])
            pltpu.async_copy(rows.at[b], acc.at[didx.at[i]], sem_s.at[b],
                             add=True)
            return carry

        lax.fori_loop(0, NCHUNK, step, 0)
        pltpu.make_async_copy(rows.at[0], acc.at[didx.at[0]],
                              sem_s.at[lax.rem(NCHUNK - 1, NBUF)]).wait()
        pltpu.make_async_copy(y_hbm.at[sidx.at[0]],
                              rows.at[0], sem.at[lax.rem(NCHUNK, NBUF)]).wait()
        plsc.subcore_barrier()

        @pl.when(sid < NS - 1)
        def _():
            pltpu.sync_copy(acc.at[pl.ds(sid * RPT, RPT)],
                            out_hbm.at[cid, pl.ds(sid * RPT, RPT)])

        @pl.when(sid == NS - 1)
        def _():
            pltpu.sync_copy(acc.at[pl.ds((NS - 1) * RPT, LAST)],
                            out_hbm.at[cid, pl.ds((NS - 1) * RPT, LAST)])

    return edge_agg


_make_edge_agg = functools.lru_cache(maxsize=None)(_make_edge_agg)


def _mm_body(x_ref, w_ref, o_ref):
    o_ref[...] = jnp.dot(x_ref[...], w_ref[...],
                         preferred_element_type=jnp.float32)


def _mm(x, w):
    din, dout = w.shape
    return pl.pallas_call(
        _mm_body,
        grid=(NBLK,),
        in_specs=[pl.BlockSpec((ROWS, din), lambda i: (i, 0)),
                  pl.BlockSpec((din, dout), lambda i: (0, 0))],
        out_specs=pl.BlockSpec((ROWS, dout), lambda i: (i, 0)),
        out_shape=jax.ShapeDtypeStruct((N, dout), jnp.float32),
    )(x, w)


def _norm_relu(a, h_ref, wroot_ref, b_ref, g_ref, be_ref):
    z = (a
         + jnp.dot(h_ref[...], wroot_ref[...],
                   preferred_element_type=jnp.float32)
         + b_ref[...])
    mu = jnp.mean(z, axis=1, keepdims=True)
    zc = z - mu
    var = jnp.mean(zc * zc, axis=1, keepdims=True)
    return jnp.maximum(zc * lax.rsqrt(var + EPS) * g_ref[...] + be_ref[...],
                       0.0)


def _layer_body(agg_ref, h_ref, wroot_ref, b_ref, g_ref, be_ref, wnext_ref,
                h_out, y_out):
    h = _norm_relu(agg_ref[0] + agg_ref[1], h_ref, wroot_ref, b_ref, g_ref,
                   be_ref)
    h_out[...] = h
    y_out[...] = jnp.dot(h, wnext_ref[...], preferred_element_type=jnp.float32)


def _layer(agg, h_prev, wroot, b, g, be, wnext):
    d = wroot.shape[1]
    dnext = wnext.shape[1]
    return pl.pallas_call(
        _layer_body,
        grid=(NBLK,),
        in_specs=[pl.BlockSpec((NC, ROWS, d), lambda i: (0, i, 0)),
                  pl.BlockSpec((ROWS, h_prev.shape[1]), lambda i: (i, 0)),
                  pl.BlockSpec(wroot.shape, lambda i: (0, 0)),
                  pl.BlockSpec((1, d), lambda i: (0, 0)),
                  pl.BlockSpec((1, d), lambda i: (0, 0)),
                  pl.BlockSpec((1, d), lambda i: (0, 0)),
                  pl.BlockSpec(wnext.shape, lambda i: (0, 0))],
        out_specs=[pl.BlockSpec((ROWS, d), lambda i: (i, 0)),
                   pl.BlockSpec((ROWS, dnext), lambda i: (i, 0))],
        out_shape=[jax.ShapeDtypeStruct((N, d), jnp.float32),
                   jax.ShapeDtypeStruct((N, dnext), jnp.float32)],
    )(agg, h_prev, wroot, b, g, be, wnext)


def _final_body(agg_ref, h_ref, wroot_ref, b_ref, g_ref, be_ref, batch_ref,
                o_ref, cnt_ref):
    i = pl.program_id(0)
    a = agg_ref[0] + agg_ref[1]
    h = _norm_relu(a, h_ref, wroot_ref, b_ref, g_ref, be_ref)
    gids = lax.broadcasted_iota(jnp.int32, (ROWS, G), 1)
    onehot = (batch_ref[...] == gids).astype(jnp.float32)
    sums = lax.dot_general(onehot, h, (((0,), (0,)), ((), ())),
                           preferred_element_type=jnp.float32)
    cnts = lax.dot_general(onehot, jnp.ones((ROWS, 1), jnp.float32),
                           (((0,), (0,)), ((), ())),
                           preferred_element_type=jnp.float32)

    @pl.when(i == 0)
    def _():
        o_ref[...] = jnp.zeros_like(o_ref)
        cnt_ref[...] = jnp.zeros_like(cnt_ref)

    o_ref[...] += sums
    cnt_ref[...] += cnts

    @pl.when(i == NBLK - 1)
    def _():
        o_ref[...] = o_ref[...] / jnp.maximum(cnt_ref[...], 1.0)


def _final(agg, h_prev, wroot, b, g, be, batch2d):
    d = wroot.shape[1]
    return pl.pallas_call(
        _final_body,
        grid=(NBLK,),
        in_specs=[pl.BlockSpec((NC, ROWS, d), lambda i: (0, i, 0)),
                  pl.BlockSpec((ROWS, h_prev.shape[1]), lambda i: (i, 0)),
                  pl.BlockSpec(wroot.shape, lambda i: (0, 0)),
                  pl.BlockSpec((1, d), lambda i: (0, 0)),
                  pl.BlockSpec((1, d), lambda i: (0, 0)),
                  pl.BlockSpec((1, d), lambda i: (0, 0)),
                  pl.BlockSpec((ROWS, 1), lambda i: (i, 0))],
        out_specs=pl.BlockSpec((G, d), lambda i: (0, 0)),
        out_shape=jax.ShapeDtypeStruct((G, d), jnp.float32),
        scratch_shapes=[pltpu.VMEM((G, 1), jnp.float32)],
    )(agg, h_prev, wroot, b, g, be, batch2d)


def kernel(x, edge_index, batch,
           W_rel1, W_root1, b1, g1, be1,
           W_rel2, W_root2, b2, g2, be2,
           W_rel3, W_root3, b3, g3, be3):
    src_r = edge_index[0].astype(jnp.int32).reshape(NW, NCHUNK, CHUNK)
    dst_r = edge_index[1].astype(jnp.int32).reshape(NW, NCHUNK, CHUNK)
    z128 = jnp.zeros((NPAD, 128), jnp.float32)
    batch2d = batch.astype(jnp.int32).reshape(N, 1)
    b1r, g1r, be1r = b1.reshape(1, -1), g1.reshape(1, -1), be1.reshape(1, -1)
    b2r, g2r, be2r = b2.reshape(1, -1), g2.reshape(1, -1), be2.reshape(1, -1)
    b3r, g3r, be3r = b3.reshape(1, -1), g3.reshape(1, -1), be3.reshape(1, -1)

    z64 = jnp.zeros((NPAD, 64), jnp.float32)

    y1 = _mm(x, W_rel1)
    a1 = _make_edge_agg(128)(y1, src_r, dst_r, z128)
    h1, y2 = _layer(a1, x, W_root1, b1r, g1r, be1r, W_rel2)
    a2 = _make_edge_agg(128)(y2, src_r, dst_r, z128)
    h2, y3 = _layer(a2, h1, W_root2, b2r, g2r, be2r, W_rel3)
    a3 = _make_edge_agg(64)(y3, src_r, dst_r, z64)
    return _final(a3, h2, W_root3, b3r, g3r, be3r, batch2d)


# speedup vs baseline: 12.6687x; 1.3069x over previous
import functools

import jax
import jax.numpy as jnp
from jax import lax
from jax.experimental import pallas as pl
from jax.experimental.pallas import tpu as pltpu
from jax.experimental.pallas import tpu_sc as plsc

N = 10000
E = 320000
G = 64
EPS = 1e-5

NC = 2
NS = 16
NW = NC * NS
EPW = E // NW
CHUNK = 50
NCHUNK = EPW // CHUNK
RPT = 632
NPAD = RPT * NS
LAST = N - RPT * (NS - 1)
NBUF = 4

ROWS = 2000
NBLK = N // ROWS


def _make_edge_agg(D):
    mesh = plsc.VectorSubcoreMesh(
        core_axis_name="c", subcore_axis_name="s",
        num_cores=NC, num_subcores=NS)

    @functools.partial(
        pl.kernel,
        out_type=jax.ShapeDtypeStruct((NC, N, D), jnp.float32),
        mesh=mesh,
        compiler_params=pltpu.CompilerParams(use_tc_tiling_on_sc=False),
        scratch_types=[
            pltpu.VMEM((NCHUNK, CHUNK), jnp.int32),
            pltpu.VMEM((NCHUNK, CHUNK), jnp.int32),
            pltpu.VMEM((NBUF, CHUNK, D), jnp.float32),
            pltpu.VMEM_SHARED((NPAD, D), jnp.float32),
            pltpu.SemaphoreType.DMA((NBUF,)),
            pltpu.SemaphoreType.DMA((NBUF,)),
        ],
    )
    def edge_agg(y_hbm, src_hbm, dst_hbm, zero_hbm, out_hbm,
                 sidx, didx, rows, acc, sem, sem_s):
        cid = lax.axis_index("c")
        sid = lax.axis_index("s")
        wid = sid * NC + cid
        pltpu.sync_copy(src_hbm.at[wid], sidx)
        pltpu.sync_copy(dst_hbm.at[wid], didx)
        pltpu.sync_copy(zero_hbm.at[pl.ds(sid * RPT, RPT)],
                        acc.at[pl.ds(sid * RPT, RPT)])
        plsc.subcore_barrier()

        def prime(k, carry):
            pltpu.async_copy(y_hbm.at[sidx.at[k]], rows.at[k], sem.at[k])
            return carry

        lax.fori_loop(0, NBUF - 1, prime, 0)

        def step(i, carry):
            b = lax.rem(i, NBUF)
            nb = lax.rem(i + NBUF - 1, NBUF)

            @pl.when(i >= 1)
            def _():
                pltpu.make_async_copy(rows.at[nb], acc.at[didx.at[0]],
                                      sem_s.at[nb]).wait()

            nxt = lax.rem(i + NBUF - 1, NCHUNK)
            pltpu.async_copy(y_hbm.at[sidx.at[nxt]], rows.at[nb], sem.at[nb])
            pltpu.make_async_copy(y_hbm.at[sidx.at[i]], rows.at[b],
                                  sem.at[b]).wait()
            pltpu.async_copy(rows.at[b], acc.at[didx.at[i]], sem_s.at[b],
                             add=True)
            return carry

        lax.fori_loop(0, NCHUNK, step, 0)
        pltpu.make_async_copy(rows.at[0], acc.at[didx.at[0]],
                              sem_s.at[lax.rem(NCHUNK - 1, NBUF)]).wait()

        def drain(k, carry):
            pltpu.make_async_copy(y_hbm.at[sidx.at[0]], rows.at[0],
                                  sem.at[lax.rem(NCHUNK + k, NBUF)]).wait()
            return carry

        lax.fori_loop(0, NBUF - 1, drain, 0)
        plsc.subcore_barrier()

        @pl.when(sid < NS - 1)
        def _():
            pltpu.sync_copy(acc.at[pl.ds(sid * RPT, RPT)],
                            out_hbm.at[cid, pl.ds(sid * RPT, RPT)])

        @pl.when(sid == NS - 1)
        def _():
            pltpu.sync_copy(acc.at[pl.ds((NS - 1) * RPT, LAST)],
                            out_hbm.at[cid, pl.ds((NS - 1) * RPT, LAST)])

    return edge_agg


_make_edge_agg = functools.lru_cache(maxsize=None)(_make_edge_agg)


def _mm_body(x_ref, w_ref, o_ref):
    o_ref[...] = jnp.dot(x_ref[...], w_ref[...],
                         preferred_element_type=jnp.float32)


def _mm(x, w):
    din, dout = w.shape
    return pl.pallas_call(
        _mm_body,
        grid=(NBLK,),
        in_specs=[pl.BlockSpec((ROWS, din), lambda i: (i, 0)),
                  pl.BlockSpec((din, dout), lambda i: (0, 0))],
        out_specs=pl.BlockSpec((ROWS, dout), lambda i: (i, 0)),
        out_shape=jax.ShapeDtypeStruct((N, dout), jnp.float32),
    )(x, w)


def _norm_relu(a, h_ref, wroot_ref, b_ref, g_ref, be_ref):
    z = (a
         + jnp.dot(h_ref[...], wroot_ref[...],
                   preferred_element_type=jnp.float32)
         + b_ref[...])
    mu = jnp.mean(z, axis=1, keepdims=True)
    zc = z - mu
    var = jnp.mean(zc * zc, axis=1, keepdims=True)
    return jnp.maximum(zc * lax.rsqrt(var + EPS) * g_ref[...] + be_ref[...],
                       0.0)


def _layer_body(agg_ref, h_ref, wroot_ref, b_ref, g_ref, be_ref, wnext_ref,
                h_out, y_out):
    h = _norm_relu(agg_ref[0] + agg_ref[1], h_ref, wroot_ref, b_ref, g_ref,
                   be_ref)
    h_out[...] = h
    y_out[...] = jnp.dot(h, wnext_ref[...], preferred_element_type=jnp.float32)


def _layer(agg, h_prev, wroot, b, g, be, wnext):
    d = wroot.shape[1]
    dnext = wnext.shape[1]
    return pl.pallas_call(
        _layer_body,
        grid=(NBLK,),
        in_specs=[pl.BlockSpec((NC, ROWS, d), lambda i: (0, i, 0)),
                  pl.BlockSpec((ROWS, h_prev.shape[1]), lambda i: (i, 0)),
                  pl.BlockSpec(wroot.shape, lambda i: (0, 0)),
                  pl.BlockSpec((1, d), lambda i: (0, 0)),
                  pl.BlockSpec((1, d), lambda i: (0, 0)),
                  pl.BlockSpec((1, d), lambda i: (0, 0)),
                  pl.BlockSpec(wnext.shape, lambda i: (0, 0))],
        out_specs=[pl.BlockSpec((ROWS, d), lambda i: (i, 0)),
                   pl.BlockSpec((ROWS, dnext), lambda i: (i, 0))],
        out_shape=[jax.ShapeDtypeStruct((N, d), jnp.float32),
                   jax.ShapeDtypeStruct((N, dnext), jnp.float32)],
    )(agg, h_prev, wroot, b, g, be, wnext)


def _final_body(agg_ref, h_ref, wroot_ref, b_ref, g_ref, be_ref, batch_ref,
                o_ref, cnt_ref):
    i = pl.program_id(0)
    a = agg_ref[0] + agg_ref[1]
    h = _norm_relu(a, h_ref, wroot_ref, b_ref, g_ref, be_ref)
    gids = lax.broadcasted_iota(jnp.int32, (ROWS, G), 1)
    onehot = (batch_ref[...] == gids).astype(jnp.float32)
    sums = lax.dot_general(onehot, h, (((0,), (0,)), ((), ())),
                           preferred_element_type=jnp.float32)
    cnts = lax.dot_general(onehot, jnp.ones((ROWS, 1), jnp.float32),
                           (((0,), (0,)), ((), ())),
                           preferred_element_type=jnp.float32)

    @pl.when(i == 0)
    def _():
        o_ref[...] = jnp.zeros_like(o_ref)
        cnt_ref[...] = jnp.zeros_like(cnt_ref)

    o_ref[...] += sums
    cnt_ref[...] += cnts

    @pl.when(i == NBLK - 1)
    def _():
        o_ref[...] = o_ref[...] / jnp.maximum(cnt_ref[...], 1.0)


def _final(agg, h_prev, wroot, b, g, be, batch2d):
    d = wroot.shape[1]
    return pl.pallas_call(
        _final_body,
        grid=(NBLK,),
        in_specs=[pl.BlockSpec((NC, ROWS, d), lambda i: (0, i, 0)),
                  pl.BlockSpec((ROWS, h_prev.shape[1]), lambda i: (i, 0)),
                  pl.BlockSpec(wroot.shape, lambda i: (0, 0)),
                  pl.BlockSpec((1, d), lambda i: (0, 0)),
                  pl.BlockSpec((1, d), lambda i: (0, 0)),
                  pl.BlockSpec((1, d), lambda i: (0, 0)),
                  pl.BlockSpec((ROWS, 1), lambda i: (i, 0))],
        out_specs=pl.BlockSpec((G, d), lambda i: (0, 0)),
        out_shape=jax.ShapeDtypeStruct((G, d), jnp.float32),
        scratch_shapes=[pltpu.VMEM((G, 1), jnp.float32)],
    )(agg, h_prev, wroot, b, g, be, batch2d)


def kernel(x, edge_index, batch,
           W_rel1, W_root1, b1, g1, be1,
           W_rel2, W_root2, b2, g2, be2,
           W_rel3, W_root3, b3, g3, be3):
    src_r = edge_index[0].astype(jnp.int32).reshape(NW, NCHUNK, CHUNK)
    dst_r = edge_index[1].astype(jnp.int32).reshape(NW, NCHUNK, CHUNK)
    z128 = jnp.zeros((NPAD, 128), jnp.float32)
    batch2d = batch.astype(jnp.int32).reshape(N, 1)
    b1r, g1r, be1r = b1.reshape(1, -1), g1.reshape(1, -1), be1.reshape(1, -1)
    b2r, g2r, be2r = b2.reshape(1, -1), g2.reshape(1, -1), be2.reshape(1, -1)
    b3r, g3r, be3r = b3.reshape(1, -1), g3.reshape(1, -1), be3.reshape(1, -1)

    z64 = jnp.zeros((NPAD, 64), jnp.float32)

    y1 = _mm(x, W_rel1)
    a1 = _make_edge_agg(128)(y1, src_r, dst_r, z128)
    h1, y2 = _layer(a1, x, W_root1, b1r, g1r, be1r, W_rel2)
    a2 = _make_edge_agg(128)(y2, src_r, dst_r, z128)
    h2, y3 = _layer(a2, h1, W_root2, b2r, g2r, be2r, W_rel3)
    a3 = _make_edge_agg(64)(y3, src_r, dst_r, z64)
    return _final(a3, h2, W_root3, b3r, g3r, be3r, batch2d)


# speedup vs baseline: 14.0158x; 1.1063x over previous
import functools

import jax
import jax.numpy as jnp
from jax import lax
from jax.experimental import pallas as pl
from jax.experimental.pallas import tpu as pltpu
from jax.experimental.pallas import tpu_sc as plsc

N = 10000
E = 320000
G = 64
EPS = 1e-5

NC = 2
NS = 16
NW = NC * NS
EPW = E // NW
RPT = 632
NPAD = RPT * NS
LAST = N - RPT * (NS - 1)

ROWS = 1000
NBLK = N // ROWS


def _make_edge_agg(D):
    NBUF = 5
    CHUNK = 40 if D == 128 else 80
    NCHUNK = EPW // CHUNK
    mesh = plsc.VectorSubcoreMesh(
        core_axis_name="c", subcore_axis_name="s",
        num_cores=NC, num_subcores=NS)

    @functools.partial(
        pl.kernel,
        out_type=jax.ShapeDtypeStruct((NC, N, D), jnp.float32),
        mesh=mesh,
        compiler_params=pltpu.CompilerParams(use_tc_tiling_on_sc=False),
        scratch_types=[
            pltpu.VMEM((NCHUNK, CHUNK), jnp.int32),
            pltpu.VMEM((NCHUNK, CHUNK), jnp.int32),
            pltpu.VMEM((NBUF, CHUNK, D), jnp.float32),
            pltpu.VMEM_SHARED((NPAD, D), jnp.float32),
            pltpu.SemaphoreType.DMA((NBUF,)),
            pltpu.SemaphoreType.DMA((NBUF,)),
            pltpu.SemaphoreType.DMA,
            pltpu.SemaphoreType.DMA,
        ],
    )
    def edge_agg(y_hbm, src_hbm, dst_hbm, zero_hbm, out_hbm,
                 sidx, didx, rows, acc, sem, sem_s, sem_z, sem_i):
        cid = lax.axis_index("c")
        sid = lax.axis_index("s")
        wid = sid * NC + cid
        zero_cp = pltpu.make_async_copy(zero_hbm.at[pl.ds(sid * RPT, RPT)],
                                        acc.at[pl.ds(sid * RPT, RPT)], sem_z)
        zero_cp.start()
        didx_cp = pltpu.make_async_copy(dst_hbm.at[wid], didx, sem_i)
        didx_cp.start()
        pltpu.sync_copy(src_hbm.at[wid], sidx)

        def prime(k, carry):
            pltpu.async_copy(y_hbm.at[sidx.at[k]], rows.at[k], sem.at[k])
            return carry

        lax.fori_loop(0, NBUF - 1, prime, 0)
        didx_cp.wait()
        zero_cp.wait()
        plsc.subcore_barrier()

        def step(i, carry):
            b = lax.rem(i, NBUF)
            nb = lax.rem(i + NBUF - 1, NBUF)

            @pl.when(i >= 1)
            def _():
                pltpu.make_async_copy(rows.at[nb], acc.at[didx.at[0]],
                                      sem_s.at[nb]).wait()

            nxt = lax.rem(i + NBUF - 1, NCHUNK)
            pltpu.async_copy(y_hbm.at[sidx.at[nxt]], rows.at[nb], sem.at[nb])
            pltpu.make_async_copy(y_hbm.at[sidx.at[i]], rows.at[b],
                                  sem.at[b]).wait()
            pltpu.async_copy(rows.at[b], acc.at[didx.at[i]], sem_s.at[b],
                             add=True)
            return carry

        lax.fori_loop(0, NCHUNK, step, 0)
        pltpu.make_async_copy(rows.at[0], acc.at[didx.at[0]],
                              sem_s.at[lax.rem(NCHUNK - 1, NBUF)]).wait()

        def drain(k, carry):
            pltpu.make_async_copy(y_hbm.at[sidx.at[0]], rows.at[0],
                                  sem.at[lax.rem(NCHUNK + k, NBUF)]).wait()
            return carry

        lax.fori_loop(0, NBUF - 1, drain, 0)
        plsc.subcore_barrier()

        @pl.when(sid < NS - 1)
        def _():
            pltpu.sync_copy(acc.at[pl.ds(sid * RPT, RPT)],
                            out_hbm.at[cid, pl.ds(sid * RPT, RPT)])

        @pl.when(sid == NS - 1)
        def _():
            pltpu.sync_copy(acc.at[pl.ds((NS - 1) * RPT, LAST)],
                            out_hbm.at[cid, pl.ds((NS - 1) * RPT, LAST)])

    return edge_agg


_make_edge_agg = functools.lru_cache(maxsize=None)(_make_edge_agg)


def _mm_body(x_ref, w_ref, o_ref):
    o_ref[...] = jnp.dot(x_ref[...], w_ref[...],
                         preferred_element_type=jnp.float32)


def _mm(x, w):
    din, dout = w.shape
    return pl.pallas_call(
        _mm_body,
        grid=(NBLK,),
        in_specs=[pl.BlockSpec((ROWS, din), lambda i: (i, 0)),
                  pl.BlockSpec((din, dout), lambda i: (0, 0))],
        out_specs=pl.BlockSpec((ROWS, dout), lambda i: (i, 0)),
        out_shape=jax.ShapeDtypeStruct((N, dout), jnp.float32),
    )(x, w)


def _norm_relu(a, h_ref, wroot_ref, b_ref, g_ref, be_ref):
    z = (a
         + jnp.dot(h_ref[...], wroot_ref[...],
                   preferred_element_type=jnp.float32)
         + b_ref[...])
    mu = jnp.mean(z, axis=1, keepdims=True)
    zc = z - mu
    var = jnp.mean(zc * zc, axis=1, keepdims=True)
    return jnp.maximum(zc * lax.rsqrt(var + EPS) * g_ref[...] + be_ref[...],
                       0.0)


def _layer_body(agg_ref, h_ref, wroot_ref, b_ref, g_ref, be_ref, wnext_ref,
                h_out, y_out):
    h = _norm_relu(agg_ref[0] + agg_ref[1], h_ref, wroot_ref, b_ref, g_ref,
                   be_ref)
    h_out[...] = h
    y_out[...] = jnp.dot(h, wnext_ref[...], preferred_element_type=jnp.float32)


def _layer(agg, h_prev, wroot, b, g, be, wnext):
    d = wroot.shape[1]
    dnext = wnext.shape[1]
    return pl.pallas_call(
        _layer_body,
        grid=(NBLK,),
        in_specs=[pl.BlockSpec((NC, ROWS, d), lambda i: (0, i, 0)),
                  pl.BlockSpec((ROWS, h_prev.shape[1]), lambda i: (i, 0)),
                  pl.BlockSpec(wroot.shape, lambda i: (0, 0)),
                  pl.BlockSpec((1, d), lambda i: (0, 0)),
                  pl.BlockSpec((1, d), lambda i: (0, 0)),
                  pl.BlockSpec((1, d), lambda i: (0, 0)),
                  pl.BlockSpec(wnext.shape, lambda i: (0, 0))],
        out_specs=[pl.BlockSpec((ROWS, d), lambda i: (i, 0)),
                   pl.BlockSpec((ROWS, dnext), lambda i: (i, 0))],
        out_shape=[jax.ShapeDtypeStruct((N, d), jnp.float32),
                   jax.ShapeDtypeStruct((N, dnext), jnp.float32)],
    )(agg, h_prev, wroot, b, g, be, wnext)


def _final_body(agg_ref, h_ref, wroot_ref, b_ref, g_ref, be_ref, batch_ref,
                o_ref, cnt_ref):
    i = pl.program_id(0)
    a = agg_ref[0] + agg_ref[1]
    h = _norm_relu(a, h_ref, wroot_ref, b_ref, g_ref, be_ref)
    gids = lax.broadcasted_iota(jnp.int32, (ROWS, G), 1)
    onehot = (batch_ref[...] == gids).astype(jnp.float32)
    sums = lax.dot_general(onehot, h, (((0,), (0,)), ((), ())),
                           preferred_element_type=jnp.float32)
    cnts = lax.dot_general(onehot, jnp.ones((ROWS, 1), jnp.float32),
                           (((0,), (0,)), ((), ())),
                           preferred_element_type=jnp.float32)

    @pl.when(i == 0)
    def _():
        o_ref[...] = jnp.zeros_like(o_ref)
        cnt_ref[...] = jnp.zeros_like(cnt_ref)

    o_ref[...] += sums
    cnt_ref[...] += cnts

    @pl.when(i == NBLK - 1)
    def _():
        o_ref[...] = o_ref[...] / jnp.maximum(cnt_ref[...], 1.0)


def _final(agg, h_prev, wroot, b, g, be, batch2d):
    d = wroot.shape[1]
    return pl.pallas_call(
        _final_body,
        grid=(NBLK,),
        in_specs=[pl.BlockSpec((NC, ROWS, d), lambda i: (0, i, 0)),
                  pl.BlockSpec((ROWS, h_prev.shape[1]), lambda i: (i, 0)),
                  pl.BlockSpec(wroot.shape, lambda i: (0, 0)),
                  pl.BlockSpec((1, d), lambda i: (0, 0)),
                  pl.BlockSpec((1, d), lambda i: (0, 0)),
                  pl.BlockSpec((1, d), lambda i: (0, 0)),
                  pl.BlockSpec((ROWS, 1), lambda i: (i, 0))],
        out_specs=pl.BlockSpec((G, d), lambda i: (0, 0)),
        out_shape=jax.ShapeDtypeStruct((G, d), jnp.float32),
        scratch_shapes=[pltpu.VMEM((G, 1), jnp.float32)],
    )(agg, h_prev, wroot, b, g, be, batch2d)


def kernel(x, edge_index, batch,
           W_rel1, W_root1, b1, g1, be1,
           W_rel2, W_root2, b2, g2, be2,
           W_rel3, W_root3, b3, g3, be3):
    src_r = edge_index[0].astype(jnp.int32).reshape(NW, EPW // 40, 40)
    dst_r = edge_index[1].astype(jnp.int32).reshape(NW, EPW // 40, 40)
    src_r64 = edge_index[0].astype(jnp.int32).reshape(NW, EPW // 80, 80)
    dst_r64 = edge_index[1].astype(jnp.int32).reshape(NW, EPW // 80, 80)
    z128 = jnp.zeros((NPAD, 128), jnp.float32)
    batch2d = batch.astype(jnp.int32).reshape(N, 1)
    b1r, g1r, be1r = b1.reshape(1, -1), g1.reshape(1, -1), be1.reshape(1, -1)
    b2r, g2r, be2r = b2.reshape(1, -1), g2.reshape(1, -1), be2.reshape(1, -1)
    b3r, g3r, be3r = b3.reshape(1, -1), g3.reshape(1, -1), be3.reshape(1, -1)

    z64 = jnp.zeros((NPAD, 64), jnp.float32)

    y1 = _mm(x, W_rel1)
    a1 = _make_edge_agg(128)(y1, src_r, dst_r, z128)
    h1, y2 = _layer(a1, x, W_root1, b1r, g1r, be1r, W_rel2)
    a2 = _make_edge_agg(128)(y2, src_r, dst_r, z128)
    h2, y3 = _layer(a2, h1, W_root2, b2r, g2r, be2r, W_rel3)
    a3 = _make_edge_agg(64)(y3, src_r64, dst_r64, z64)
    return _final(a3, h2, W_root3, b3r, g3r, be3r, batch2d)


# speedup vs baseline: 14.4823x; 1.0333x over previous
import functools

import jax
import jax.numpy as jnp
from jax import lax
from jax.experimental import pallas as pl
from jax.experimental.pallas import tpu as pltpu
from jax.experimental.pallas import tpu_sc as plsc

N = 10000
E = 320000
G = 64
EPS = 1e-5

NC = 2
NS = 16
NW = NC * NS
EPW = E // NW
RPT = 632
NPAD = RPT * NS
LAST = N - RPT * (NS - 1)

ROWS = 2000
NBLK = N // ROWS


def _make_edge_agg(D):
    NBUF = 5
    CHUNK = 40 if D == 128 else 80
    NCHUNK = EPW // CHUNK
    mesh = plsc.VectorSubcoreMesh(
        core_axis_name="c", subcore_axis_name="s",
        num_cores=NC, num_subcores=NS)

    @functools.partial(
        pl.kernel,
        out_type=jax.ShapeDtypeStruct((NC, N, D), jnp.float32),
        mesh=mesh,
        compiler_params=pltpu.CompilerParams(use_tc_tiling_on_sc=False),
        scratch_types=[
            pltpu.VMEM((NCHUNK, CHUNK), jnp.int32),
            pltpu.VMEM((NCHUNK, CHUNK), jnp.int32),
            pltpu.VMEM((NBUF, CHUNK, D), jnp.float32),
            pltpu.VMEM_SHARED((NPAD, D), jnp.float32),
            pltpu.SemaphoreType.DMA((NBUF,)),
            pltpu.SemaphoreType.DMA((NBUF,)),
            pltpu.SemaphoreType.DMA,
            pltpu.SemaphoreType.DMA,
        ],
    )
    def edge_agg(y_hbm, src_hbm, dst_hbm, zero_hbm, out_hbm,
                 sidx, didx, rows, acc, sem, sem_s, sem_z, sem_i):
        cid = lax.axis_index("c")
        sid = lax.axis_index("s")
        wid = sid * NC + cid
        zero_cp = pltpu.make_async_copy(zero_hbm.at[pl.ds(sid * RPT, RPT)],
                                        acc.at[pl.ds(sid * RPT, RPT)], sem_z)
        zero_cp.start()
        didx_cp = pltpu.make_async_copy(dst_hbm.at[wid], didx, sem_i)
        didx_cp.start()
        pltpu.sync_copy(src_hbm.at[wid], sidx)

        def prime(k, carry):
            pltpu.async_copy(y_hbm.at[sidx.at[k]], rows.at[k], sem.at[k])
            return carry

        lax.fori_loop(0, NBUF - 1, prime, 0)
        didx_cp.wait()
        zero_cp.wait()
        plsc.subcore_barrier()

        def step(i, carry):
            b = lax.rem(i, NBUF)
            nb = lax.rem(i + NBUF - 1, NBUF)

            @pl.when(i >= 1)
            def _():
                pltpu.make_async_copy(rows.at[nb], acc.at[didx.at[0]],
                                      sem_s.at[nb]).wait()

            nxt = lax.rem(i + NBUF - 1, NCHUNK)
            pltpu.async_copy(y_hbm.at[sidx.at[nxt]], rows.at[nb], sem.at[nb])
            pltpu.make_async_copy(y_hbm.at[sidx.at[i]], rows.at[b],
                                  sem.at[b]).wait()
            pltpu.async_copy(rows.at[b], acc.at[didx.at[i]], sem_s.at[b],
                             add=True)
            return carry

        lax.fori_loop(0, NCHUNK, step, 0)
        pltpu.make_async_copy(rows.at[0], acc.at[didx.at[0]],
                              sem_s.at[lax.rem(NCHUNK - 1, NBUF)]).wait()

        def drain(k, carry):
            pltpu.make_async_copy(y_hbm.at[sidx.at[0]], rows.at[0],
                                  sem.at[lax.rem(NCHUNK + k, NBUF)]).wait()
            return carry

        lax.fori_loop(0, NBUF - 1, drain, 0)
        plsc.subcore_barrier()

        @pl.when(sid < NS - 1)
        def _():
            pltpu.sync_copy(acc.at[pl.ds(sid * RPT, RPT)],
                            out_hbm.at[cid, pl.ds(sid * RPT, RPT)])

        @pl.when(sid == NS - 1)
        def _():
            pltpu.sync_copy(acc.at[pl.ds((NS - 1) * RPT, LAST)],
                            out_hbm.at[cid, pl.ds((NS - 1) * RPT, LAST)])

    return edge_agg


_make_edge_agg = functools.lru_cache(maxsize=None)(_make_edge_agg)


def _mm_body(x_ref, w_ref, o_ref):
    o_ref[...] = jnp.dot(x_ref[...], w_ref[...],
                         preferred_element_type=jnp.float32)


def _mm(x, w):
    din, dout = w.shape
    return pl.pallas_call(
        _mm_body,
        grid=(NBLK,),
        in_specs=[pl.BlockSpec((ROWS, din), lambda i: (i, 0)),
                  pl.BlockSpec((din, dout), lambda i: (0, 0))],
        out_specs=pl.BlockSpec((ROWS, dout), lambda i: (i, 0)),
        out_shape=jax.ShapeDtypeStruct((N, dout), jnp.float32),
    )(x, w)


def _norm_relu(a, h_ref, wroot_ref, b_ref, g_ref, be_ref):
    z = (a
         + jnp.dot(h_ref[...], wroot_ref[...],
                   preferred_element_type=jnp.float32)
         + b_ref[...])
    mu = jnp.mean(z, axis=1, keepdims=True)
    zc = z - mu
    var = jnp.mean(zc * zc, axis=1, keepdims=True)
    return jnp.maximum(zc * lax.rsqrt(var + EPS) * g_ref[...] + be_ref[...],
                       0.0)


def _layer_body(agg_ref, h_ref, wroot_ref, b_ref, g_ref, be_ref, wnext_ref,
                h_out, y_out):
    h = _norm_relu(agg_ref[0] + agg_ref[1], h_ref, wroot_ref, b_ref, g_ref,
                   be_ref)
    h_out[...] = h
    y_out[...] = jnp.dot(h, wnext_ref[...], preferred_element_type=jnp.float32)


def _layer(agg, h_prev, wroot, b, g, be, wnext):
    d = wroot.shape[1]
    dnext = wnext.shape[1]
    return pl.pallas_call(
        _layer_body,
        grid=(NBLK,),
        in_specs=[pl.BlockSpec((NC, ROWS, d), lambda i: (0, i, 0)),
                  pl.BlockSpec((ROWS, h_prev.shape[1]), lambda i: (i, 0)),
                  pl.BlockSpec(wroot.shape, lambda i: (0, 0)),
                  pl.BlockSpec((1, d), lambda i: (0, 0)),
                  pl.BlockSpec((1, d), lambda i: (0, 0)),
                  pl.BlockSpec((1, d), lambda i: (0, 0)),
                  pl.BlockSpec(wnext.shape, lambda i: (0, 0))],
        out_specs=[pl.BlockSpec((ROWS, d), lambda i: (i, 0)),
                   pl.BlockSpec((ROWS, dnext), lambda i: (i, 0))],
        out_shape=[jax.ShapeDtypeStruct((N, d), jnp.float32),
                   jax.ShapeDtypeStruct((N, dnext), jnp.float32)],
    )(agg, h_prev, wroot, b, g, be, wnext)


def _final_body(agg_ref, h_ref, wroot_ref, b_ref, g_ref, be_ref, batch_ref,
                o_ref, cnt_ref):
    i = pl.program_id(0)
    a = agg_ref[0] + agg_ref[1]
    h = _norm_relu(a, h_ref, wroot_ref, b_ref, g_ref, be_ref)
    gids = lax.broadcasted_iota(jnp.int32, (ROWS, G), 1)
    onehot = (batch_ref[...] == gids).astype(jnp.float32)
    sums = lax.dot_general(onehot, h, (((0,), (0,)), ((), ())),
                           preferred_element_type=jnp.float32)
    cnts = lax.dot_general(onehot, jnp.ones((ROWS, 1), jnp.float32),
                           (((0,), (0,)), ((), ())),
                           preferred_element_type=jnp.float32)

    @pl.when(i == 0)
    def _():
        o_ref[...] = jnp.zeros_like(o_ref)
        cnt_ref[...] = jnp.zeros_like(cnt_ref)

    o_ref[...] += sums
    cnt_ref[...] += cnts

    @pl.when(i == NBLK - 1)
    def _():
        o_ref[...] = o_ref[...] / jnp.maximum(cnt_ref[...], 1.0)


def _final(agg, h_prev, wroot, b, g, be, batch2d):
    d = wroot.shape[1]
    return pl.pallas_call(
        _final_body,
        grid=(NBLK,),
        in_specs=[pl.BlockSpec((NC, ROWS, d), lambda i: (0, i, 0)),
                  pl.BlockSpec((ROWS, h_prev.shape[1]), lambda i: (i, 0)),
                  pl.BlockSpec(wroot.shape, lambda i: (0, 0)),
                  pl.BlockSpec((1, d), lambda i: (0, 0)),
                  pl.BlockSpec((1, d), lambda i: (0, 0)),
                  pl.BlockSpec((1, d), lambda i: (0, 0)),
                  pl.BlockSpec((ROWS, 1), lambda i: (i, 0))],
        out_specs=pl.BlockSpec((G, d), lambda i: (0, 0)),
        out_shape=jax.ShapeDtypeStruct((G, d), jnp.float32),
        scratch_shapes=[pltpu.VMEM((G, 1), jnp.float32)],
    )(agg, h_prev, wroot, b, g, be, batch2d)


def kernel(x, edge_index, batch,
           W_rel1, W_root1, b1, g1, be1,
           W_rel2, W_root2, b2, g2, be2,
           W_rel3, W_root3, b3, g3, be3):
    src_r = edge_index[0].astype(jnp.int32).reshape(NW, EPW // 40, 40)
    dst_r = edge_index[1].astype(jnp.int32).reshape(NW, EPW // 40, 40)
    src_r64 = edge_index[0].astype(jnp.int32).reshape(NW, EPW // 80, 80)
    dst_r64 = edge_index[1].astype(jnp.int32).reshape(NW, EPW // 80, 80)
    z128 = jnp.zeros((NPAD, 128), jnp.float32)
    batch2d = batch.astype(jnp.int32).reshape(N, 1)
    b1r, g1r, be1r = b1.reshape(1, -1), g1.reshape(1, -1), be1.reshape(1, -1)
    b2r, g2r, be2r = b2.reshape(1, -1), g2.reshape(1, -1), be2.reshape(1, -1)
    b3r, g3r, be3r = b3.reshape(1, -1), g3.reshape(1, -1), be3.reshape(1, -1)

    z64 = jnp.zeros((NPAD, 64), jnp.float32)

    y1 = _mm(x, W_rel1)
    a1 = _make_edge_agg(128)(y1, src_r, dst_r, z128)
    h1, y2 = _layer(a1, x, W_root1, b1r, g1r, be1r, W_rel2)
    a2 = _make_edge_agg(128)(y2, src_r, dst_r, z128)
    h2, y3 = _layer(a2, h1, W_root2, b2r, g2r, be2r, W_rel3)
    a3 = _make_edge_agg(64)(y3, src_r64, dst_r64, z64)
    return _final(a3, h2, W_root3, b3r, g3r, be3r, batch2d)
